# baseline (speedup 1.0000x reference)
import functools

import jax
import jax.numpy as jnp
from jax import lax
from jax.experimental import pallas as pl
from jax.experimental.pallas import tpu as pltpu
from jax.experimental.pallas import tpu_sc as plsc

BATCH = 16384
HIST = 50
EMBED = 32
NC = 2
NS = 16
NW = NC * NS
NBT = BATCH // 128
NUNIT = HIST * NBT
U_PER_W = NUNIT // NW
NBUF = 4
NGROUP = U_PER_W // NBUF

_mesh = plsc.VectorSubcoreMesh(core_axis_name="c", subcore_axis_name="s")


@functools.partial(
    pl.kernel,
    mesh=_mesh,
    out_type=jax.ShapeDtypeStruct((HIST, 4, NBT, 1024), jnp.float32),
    scratch_types=[
        pltpu.VMEM((NBUF, 128), jnp.int32),
        pltpu.VMEM((NBUF, 128, EMBED), jnp.float32),
        pltpu.VMEM((NBUF, 4096), jnp.float32),
    ]
    + [pltpu.SemaphoreType.DMA] * (3 * NBUF),
    compiler_params=pltpu.CompilerParams(use_tc_tiling_on_sc=False, needs_layout_passes=False),
)
def _gather_fmt(idx_hbm, tlin_hbm, out_hbm, idx_v, rows_v, tile_v, *sems):
    sem_i = sems[0:NBUF]
    sem_g = sems[NBUF:2 * NBUF]
    sem_o = sems[2 * NBUF:]

    wid = lax.axis_index("s") * NC + lax.axis_index("c")
    u0 = wid * U_PER_W
    iota16 = lax.iota(jnp.int32, 16)
    iota128 = iota16 * 128

    def unit_hbt(u):
        uu = u0 + u
        return uu // NBT, uu % NBT

    def idx_copy(u, b):
        h, bt = unit_hbt(u)
        return pltpu.make_async_copy(
            idx_hbm.at[h, pl.ds(bt * 128, 128)], idx_v.at[b], sem_i[b])

    def gather_copy(b):
        return pltpu.make_async_copy(
            tlin_hbm.at[idx_v.at[b]], rows_v.at[b], sem_g[b])

    def out_copy(u, b, k):
        h, bt = unit_hbt(u)
        return pltpu.make_async_copy(
            tile_v.at[b, pl.ds(k * 1024, 1024)], out_hbm.at[h, k, bt], sem_o[b])

    def transpose(b):
        def body(i, carry):
            for j in range(8):
                l = i * 8 + j
                lvec = iota16 * 0 + l
                for d0 in (0, 16):
                    v = plsc.load_gather(rows_v.at[b], [lvec, iota16 + d0])
                    plsc.store_scatter(
                        tile_v.at[b], [iota128 + (d0 * 128 + l)], v)
            return carry
        lax.fori_loop(0, 16, body, 0)

    for b in range(NBUF):
        idx_copy(b, b).start()

    def group(g, first, last):
        for b in range(NBUF):
            u = g * NBUF + b
            idx_copy(u, b).wait()
            if not first:
                for k in range(4):
                    out_copy(u - NBUF, b, k).wait()
            gather_copy(b).start()
        for b in range(NBUF):
            u = g * NBUF + b
            gather_copy(b).wait()
            if not last:
                idx_copy(u + NBUF, b).start()
            transpose(b)
            for k in range(4):
                out_copy(u, b, k).start()

    group(0, True, NGROUP == 1)

    def body(g, carry):
        group(g, False, False)
        return carry

    lax.fori_loop(1, NGROUP - 1, body, 0)
    group(NGROUP - 1, False, True)

    for b in range(NBUF):
        for k in range(4):
            out_copy((NGROUP - 1) * NBUF + b, b, k).wait()


NOBJ = 1000000
SLAB = 512
NSLAB = NOBJ // SLAB
TAIL = NOBJ - NSLAB * SLAB
G_PER_W = NSLAB // NW


@functools.partial(
    pl.kernel,
    mesh=_mesh,
    out_type=jax.ShapeDtypeStruct((NOBJ * EMBED,), jnp.float32),
    scratch_types=[
        pltpu.VMEM((EMBED, SLAB), jnp.float32),
        pltpu.VMEM((EMBED, SLAB), jnp.float32),
        pltpu.VMEM((SLAB * EMBED,), jnp.float32),
        pltpu.VMEM((SLAB * EMBED,), jnp.float32),
    ]
    + [pltpu.SemaphoreType.DMA] * 4,
    compiler_params=pltpu.CompilerParams(use_tc_tiling_on_sc=True, needs_layout_passes=False),
)
def _detile(tab_hbm, tail_hbm, out_hbm, in_v0, in_v1, row_v0, row_v1, *sems):
    in_v = (in_v0, in_v1)
    row_v = (row_v0, row_v1)
    sem_i = sems[0:2]
    sem_o = sems[2:4]
    wid = lax.axis_index("s") * NC + lax.axis_index("c")
    iota16 = lax.iota(jnp.int32, 16)
    iota512 = iota16 * EMBED

    def in_copy(s, b, width=SLAB):
        return pltpu.make_async_copy(
            tab_hbm.at[:, pl.ds(s * SLAB, width)],
            in_v[b],
            sem_i[b])

    def out_copy(s, b, width=SLAB):
        return pltpu.make_async_copy(
            row_v[b],
            out_hbm.at[pl.ds(s * (SLAB * EMBED), width * EMBED)],
            sem_o[b])

    def transpose(b, width=SLAB):
        def body(oc, carry):
            for d in range(EMBED):
                v = plsc.load_gather(
                    in_v[b], [iota16 * 0 + d, iota16 + oc * 16])
                plsc.store_scatter(row_v[b], [iota512 + (oc * 16 * EMBED + d)], v)
            return carry
        lax.fori_loop(0, width // 16, body, 0)

    def slab_of(g):
        return wid + g * NW

    in_copy(slab_of(0), 0).start()
    in_copy(slab_of(1), 1).start()

    def step(g, b, first, last):
        in_copy(slab_of(g), b).wait()
        if not first:
            out_copy(slab_of(g - 2), b).wait()
        transpose(b)
        if not last:
            in_copy(slab_of(g + 2), b).start()
        out_copy(slab_of(g), b).start()

    step(0, 0, True, False)
    step(1, 1, True, False)

    def body(k, carry):
        g = 2 + k * 2
        step(g, 0, False, False)
        step(g + 1, 1, False, False)
        return carry

    lax.fori_loop(0, (G_PER_W - 5) // 2, body, 0)
    step(G_PER_W - 3, 0, False, False)
    step(G_PER_W - 2, 1, False, True)
    step(G_PER_W - 1, 0, False, True)
    out_copy(slab_of(G_PER_W - 2), 1).wait()
    out_copy(slab_of(G_PER_W - 1), 0).wait()

    @pl.when(wid == 0)
    def _():
        in_copy(NSLAB - 1, 0).start()
        in_copy(NSLAB - 1, 0).wait()
        transpose(0)
        out_copy(NSLAB - 1, 0).start()
        out_copy(NSLAB - 1, 0).wait()

    @pl.when(wid == 1)
    def _():
        def tail_in():
            return pltpu.make_async_copy(
                tail_hbm, row_v0.at[pl.ds(0, TAIL * EMBED)], sem_i[0])

        def tail_out():
            return pltpu.make_async_copy(
                row_v0.at[pl.ds(0, TAIL * EMBED)],
                out_hbm.at[pl.ds(NSLAB * SLAB * EMBED, TAIL * EMBED)],
                sem_o[0])

        tail_in().start()
        tail_in().wait()
        tail_out().start()
        tail_out().wait()


def kernel(object_index, table):
    idx_t = object_index.astype(jnp.int32).T
    tail = table[NSLAB * SLAB:].reshape(TAIL * EMBED)
    tlin = _detile(table.T, tail)
    out = _gather_fmt(idx_t, tlin.reshape(NOBJ, EMBED))
    out5 = out.reshape(HIST, 4, NBT, 8, 128)
    return out5.transpose(2, 4, 0, 1, 3).reshape(BATCH, HIST, EMBED)


# speedup vs baseline: 2.1722x; 2.1722x over previous
import functools

import jax
import jax.numpy as jnp
from jax import lax
from jax.experimental import pallas as pl
from jax.experimental.pallas import tpu as pltpu
from jax.experimental.pallas import tpu_sc as plsc

BATCH = 16384
HIST = 50
EMBED = 32
NC = 2
NS = 16
NW = NC * NS
NBT = BATCH // 128
NUNIT = HIST * NBT
U_PER_W = NUNIT // NW
NBUF = 4
NGROUP = U_PER_W // NBUF

_mesh = plsc.VectorSubcoreMesh(core_axis_name="c", subcore_axis_name="s")


@functools.partial(
    pl.kernel,
    mesh=_mesh,
    out_type=jax.ShapeDtypeStruct((HIST, 4, NBT, 1024), jnp.float32),
    scratch_types=[
        pltpu.VMEM((NBUF, 128), jnp.int32),
        pltpu.VMEM((NBUF, 128, EMBED), jnp.float32),
        pltpu.VMEM((NBUF, 4096), jnp.float32),
    ]
    + [pltpu.SemaphoreType.DMA] * (3 * NBUF),
    compiler_params=pltpu.CompilerParams(use_tc_tiling_on_sc=False, needs_layout_passes=False),
)
def _gather_fmt(idx_hbm, tlin_hbm, out_hbm, idx_v, rows_v, tile_v, *sems):
    sem_i = sems[0:NBUF]
    sem_g = sems[NBUF:2 * NBUF]
    sem_o = sems[2 * NBUF:]

    wid = lax.axis_index("s") * NC + lax.axis_index("c")
    u0 = wid * U_PER_W
    iota16 = lax.iota(jnp.int32, 16)
    cjs = [(iota16 + j) & 15 for j in range(16)]
    cj128 = [cjs[j] * 128 + iota16 for j in range(16)]

    def unit_hbt(u):
        uu = u0 + u
        return uu // NBT, uu % NBT

    def idx_copy(u, b):
        h, bt = unit_hbt(u)
        return pltpu.make_async_copy(
            idx_hbm.at[h, pl.ds(bt * 128, 128)], idx_v.at[b], sem_i[b])

    def gather_copy(b):
        return pltpu.make_async_copy(
            tlin_hbm.at[idx_v.at[b]], rows_v.at[b], sem_g[b])

    def out_copy(u, b, k):
        h, bt = unit_hbt(u)
        return pltpu.make_async_copy(
            tile_v.at[b, pl.ds(k * 1024, 1024)], out_hbm.at[h, k, bt], sem_o[b])

    def transpose(b):
        def body(i, carry):
            l0 = i * 16
            rowv = iota16 + l0
            for d0 in (0, 16):
                for j in range(16):
                    v = plsc.load_gather(rows_v.at[b], [rowv, cjs[j] + d0])
                    plsc.store_scatter(
                        tile_v.at[b], [cj128[j] + (d0 * 128 + l0)], v)
            return carry
        lax.fori_loop(0, 8, body, 0)

    for b in range(NBUF):
        idx_copy(b, b).start()

    def group(g, first, last):
        for b in range(NBUF):
            u = g * NBUF + b
            idx_copy(u, b).wait()
            if not first:
                for k in range(4):
                    out_copy(u - NBUF, b, k).wait()
            gather_copy(b).start()
        for b in range(NBUF):
            u = g * NBUF + b
            gather_copy(b).wait()
            if not last:
                idx_copy(u + NBUF, b).start()
            transpose(b)
            for k in range(4):
                out_copy(u, b, k).start()

    group(0, True, NGROUP == 1)

    def body(g, carry):
        group(g, False, False)
        return carry

    lax.fori_loop(1, NGROUP - 1, body, 0)
    group(NGROUP - 1, False, True)

    for b in range(NBUF):
        for k in range(4):
            out_copy((NGROUP - 1) * NBUF + b, b, k).wait()


NOBJ = 1000000
SLAB = 512
NSLAB = NOBJ // SLAB
TAIL = NOBJ - NSLAB * SLAB
G_PER_W = NSLAB // NW


@functools.partial(
    pl.kernel,
    mesh=_mesh,
    out_type=jax.ShapeDtypeStruct((NOBJ * EMBED,), jnp.float32),
    scratch_types=[
        pltpu.VMEM((EMBED, SLAB), jnp.float32),
        pltpu.VMEM((EMBED, SLAB), jnp.float32),
        pltpu.VMEM((SLAB * EMBED,), jnp.float32),
        pltpu.VMEM((SLAB * EMBED,), jnp.float32),
    ]
    + [pltpu.SemaphoreType.DMA] * 4,
    compiler_params=pltpu.CompilerParams(use_tc_tiling_on_sc=True, needs_layout_passes=False),
)
def _detile(tab_hbm, tail_hbm, out_hbm, in_v0, in_v1, row_v0, row_v1, *sems):
    in_v = (in_v0, in_v1)
    row_v = (row_v0, row_v1)
    sem_i = sems[0:2]
    sem_o = sems[2:4]
    wid = lax.axis_index("s") * NC + lax.axis_index("c")
    iota16 = lax.iota(jnp.int32, 16)
    cjs = [(iota16 + j) & 15 for j in range(16)]
    svs = [iota16 * EMBED + cjs[j] for j in range(16)]

    def in_copy(s, b, width=SLAB):
        return pltpu.make_async_copy(
            tab_hbm.at[:, pl.ds(s * SLAB, width)],
            in_v[b],
            sem_i[b])

    def out_copy(s, b, width=SLAB):
        return pltpu.make_async_copy(
            row_v[b],
            out_hbm.at[pl.ds(s * (SLAB * EMBED), width * EMBED)],
            sem_o[b])

    def transpose(b, width=SLAB):
        def body(oc, carry):
            colv = iota16 + oc * 16
            for d0 in (0, 16):
                for j in range(16):
                    v = plsc.load_gather(in_v[b], [cjs[j] + d0, colv])
                    plsc.store_scatter(
                        row_v[b], [svs[j] + (oc * (16 * EMBED) + d0)], v)
            return carry
        lax.fori_loop(0, width // 16, body, 0)

    def slab_of(g):
        return wid + g * NW

    in_copy(slab_of(0), 0).start()
    in_copy(slab_of(1), 1).start()

    def step(g, b, first, last):
        in_copy(slab_of(g), b).wait()
        if not first:
            out_copy(slab_of(g - 2), b).wait()
        transpose(b)
        if not last:
            in_copy(slab_of(g + 2), b).start()
        out_copy(slab_of(g), b).start()

    step(0, 0, True, False)
    step(1, 1, True, False)

    def body(k, carry):
        g = 2 + k * 2
        step(g, 0, False, False)
        step(g + 1, 1, False, False)
        return carry

    lax.fori_loop(0, (G_PER_W - 5) // 2, body, 0)
    step(G_PER_W - 3, 0, False, False)
    step(G_PER_W - 2, 1, False, True)
    step(G_PER_W - 1, 0, False, True)
    out_copy(slab_of(G_PER_W - 2), 1).wait()
    out_copy(slab_of(G_PER_W - 1), 0).wait()

    @pl.when(wid == 0)
    def _():
        in_copy(NSLAB - 1, 0).start()
        in_copy(NSLAB - 1, 0).wait()
        transpose(0)
        out_copy(NSLAB - 1, 0).start()
        out_copy(NSLAB - 1, 0).wait()

    @pl.when(wid == 1)
    def _():
        def tail_in():
            return pltpu.make_async_copy(
                tail_hbm, row_v0.at[pl.ds(0, TAIL * EMBED)], sem_i[0])

        def tail_out():
            return pltpu.make_async_copy(
                row_v0.at[pl.ds(0, TAIL * EMBED)],
                out_hbm.at[pl.ds(NSLAB * SLAB * EMBED, TAIL * EMBED)],
                sem_o[0])

        tail_in().start()
        tail_in().wait()
        tail_out().start()
        tail_out().wait()


def kernel(object_index, table):
    idx_t = object_index.astype(jnp.int32).T
    tail = table[NSLAB * SLAB:].reshape(TAIL * EMBED)
    tlin = _detile(table.T, tail)
    out = _gather_fmt(idx_t, tlin.reshape(NOBJ, EMBED))
    out5 = out.reshape(HIST, 4, NBT, 8, 128)
    return out5.transpose(2, 4, 0, 1, 3).reshape(BATCH, HIST, EMBED)
